# baseline (speedup 1.0000x reference)
import numpy as np

import jax
import jax.numpy as jnp
from jax import lax
from jax.experimental import pallas as pl
from jax.experimental.pallas import tpu as pltpu
from jax.experimental.pallas import tpu_sc as plsc

B, F = 16384, 100
NC, NS, L = 1, 16, 16
NW = NC * NS
SC_COLS = 4096
COLS_W = SC_COLS // NW
CHUNK = 128
NCHUNK = COLS_W // CHUNK
VPR = CHUNK // L
TC_BLK = 1024
TC_G = (B - SC_COLS) // TC_BLK

_BIAS = 113 << 8
_NTAB = 3585
_TAB_PAD = 3648


def _make_table() -> np.ndarray:
    i = np.arange(_NTAB - 1, dtype=np.int64)
    lo = (i + _BIAS) << 15
    hi = lo + (1 << 15)
    xlo = lo.astype(np.uint32).view(np.float32).astype(np.float64)
    xhi = hi.astype(np.uint32).view(np.float32).astype(np.float64)
    f = lambda x: x * np.log(x) - x
    tab = (f(xhi) - f(xlo)) / (xhi - xlo)
    tab = np.append(tab, 0.0)
    return np.pad(tab, (0, _TAB_PAD - _NTAB)).astype(np.float32)


_TABLE = _make_table()


def _bce_body(yh_hbm, y_hbm, tab_hbm, out_hbm,
              tab_v, yh0, y0, yh1, y1, out_v,
              sem_t, sem_a, sem_b, sem_o):
    c = lax.axis_index("c")
    s = lax.axis_index("s")
    wid = c * NS + s
    col0 = wid * COLS_W

    pltpu.make_async_copy(tab_hbm, tab_v, sem_t).start()

    bufs = ((yh0, y0), (yh1, y1))

    def start(ci, slot):
        c0 = col0 + ci * CHUNK
        yv, tv = bufs[slot]
        pltpu.make_async_copy(yh_hbm.at[:, pl.ds(c0, CHUNK)], yv, sem_a).start()
        pltpu.make_async_copy(y_hbm.at[:, pl.ds(c0, CHUNK)], tv, sem_b).start()

    def wait(ci, slot):
        c0 = col0 + ci * CHUNK
        yv, tv = bufs[slot]
        pltpu.make_async_copy(yh_hbm.at[:, pl.ds(c0, CHUNK)], yv, sem_a).wait()
        pltpu.make_async_copy(y_hbm.at[:, pl.ds(c0, CHUNK)], tv, sem_b).wait()

    start(0, 0)
    pltpu.make_async_copy(tab_hbm, tab_v, sem_t).wait()

    one = jnp.float32(1.0)

    def elem(p, t, acc, cnt):
        valid = t > -0.5
        x = jnp.where(t > 0.5, p, one - p)
        x = jnp.where(valid, x, one)
        idx = (lax.bitcast_convert_type(x, jnp.int32) >> 15) - _BIAS
        acc = acc - plsc.load_gather(tab_v, [idx])
        cnt = cnt + jnp.where(valid, one, 0.0)
        return acc, cnt

    start(1, 1)

    def half(ci, slot, carry):
        acc, cnt = carry
        wait(ci, slot)
        yv, tv = bufs[slot]

        def rows(r, carry):
            acc, cnt = carry
            for v in range(VPR):
                p = yv[r, pl.ds(v * L, L)]
                t = tv[r, pl.ds(v * L, L)]
                acc, cnt = elem(p, t, acc, cnt)
            return acc, cnt

        carry = lax.fori_loop(0, F, rows, (acc, cnt))

        @pl.when(ci + 2 < NCHUNK)
        def _():
            start(ci + 2, slot)

        return carry

    def chunk_pair(i, carry):
        ci = i * 2
        carry = half(ci, 0, carry)
        return half(ci + 1, 1, carry)

    acc, cnt = lax.fori_loop(
        0, NCHUNK // 2, chunk_pair,
        (jnp.zeros((L,), jnp.float32), jnp.zeros((L,), jnp.float32)))

    out_v[pl.ds(0, L)] = acc
    out_v[pl.ds(L, L)] = cnt
    pltpu.make_async_copy(out_v, out_hbm.at[pl.ds(wid * 2 * L, 2 * L)],
                          sem_o).start()
    pltpu.make_async_copy(out_v, out_hbm.at[pl.ds(wid * 2 * L, 2 * L)],
                          sem_o).wait()


_bce_call = pl.kernel(
    _bce_body,
    mesh=plsc.VectorSubcoreMesh(core_axis_name="c", subcore_axis_name="s", num_cores=NC),
    compiler_params=pltpu.CompilerParams(
        needs_layout_passes=False, skip_device_barrier=True),
    out_type=jax.ShapeDtypeStruct((NW * 2 * L,), jnp.float32),
    scratch_types=[
        pltpu.VMEM((_TAB_PAD,), jnp.float32),
        pltpu.VMEM((F, CHUNK), jnp.float32),
        pltpu.VMEM((F, CHUNK), jnp.float32),
        pltpu.VMEM((F, CHUNK), jnp.float32),
        pltpu.VMEM((F, CHUNK), jnp.float32),
        pltpu.VMEM((2 * L,), jnp.float32),
        pltpu.SemaphoreType.DMA,
        pltpu.SemaphoreType.DMA,
        pltpu.SemaphoreType.DMA,
        pltpu.SemaphoreType.DMA,
    ],
)


def _tc_body(yh_ref, y_ref, sum_ref, cnt_ref):
    p = yh_ref[...]
    t = y_ref[...]
    valid = t > -0.5
    x = jnp.where(t > 0.5, p, 1.0 - p)
    x = jnp.where(valid, x, 1.0)
    lnx = jnp.maximum(jnp.log(x), -100.0)
    sum_ref[...] = jnp.full((1, 1, 128), -jnp.sum(lnx), jnp.float32)
    cnt_ref[...] = jnp.full((1, 1, 128), jnp.sum(jnp.where(valid, 1.0, 0.0)),
                            jnp.float32)


_tc_call = pl.pallas_call(
    _tc_body,
    grid=(TC_G,),
    in_specs=[
        pl.BlockSpec((F, TC_BLK), lambda i: (0, SC_COLS // TC_BLK + i)),
        pl.BlockSpec((F, TC_BLK), lambda i: (0, SC_COLS // TC_BLK + i)),
    ],
    out_specs=[
        pl.BlockSpec((1, 1, 128), lambda i: (i, 0, 0)),
        pl.BlockSpec((1, 1, 128), lambda i: (i, 0, 0)),
    ],
    out_shape=[
        jax.ShapeDtypeStruct((TC_G, 1, 128), jnp.float32),
        jax.ShapeDtypeStruct((TC_G, 1, 128), jnp.float32),
    ],
)


def kernel(y_hat, y):
    yt, tt = y_hat.T, y.T
    parts = _bce_call(yt, tt, jnp.asarray(_TABLE)).reshape(NW, 2, L)
    tc_sum, tc_cnt = _tc_call(yt, tt)
    total = jnp.sum(parts[:, 0]) + jnp.sum(tc_sum[:, 0, 0])
    n = jnp.sum(parts[:, 1]) + jnp.sum(tc_cnt[:, 0, 0])
    return total / n


# speedup vs baseline: 1.0431x; 1.0431x over previous
import numpy as np

import jax
import jax.numpy as jnp
from jax import lax
from jax.experimental import pallas as pl
from jax.experimental.pallas import tpu as pltpu
from jax.experimental.pallas import tpu_sc as plsc

B, F = 16384, 100
NC, NS, L = 2, 16, 16
NW = NC * NS
SC_COLS = 8192
COLS_W = SC_COLS // NW
CHUNK = 128
NCHUNK = COLS_W // CHUNK
VPR = CHUNK // L
TC_BLK = 1024
TC_G = (B - SC_COLS) // TC_BLK

_BIAS = 113 << 8
_NTAB = 3585
_TAB_PAD = 3648


def _make_table() -> np.ndarray:
    i = np.arange(_NTAB - 1, dtype=np.int64)
    lo = (i + _BIAS) << 15
    hi = lo + (1 << 15)
    xlo = lo.astype(np.uint32).view(np.float32).astype(np.float64)
    xhi = hi.astype(np.uint32).view(np.float32).astype(np.float64)
    f = lambda x: x * np.log(x) - x
    tab = (f(xhi) - f(xlo)) / (xhi - xlo)
    tab = np.append(tab, 0.0)
    return np.pad(tab, (0, _TAB_PAD - _NTAB)).astype(np.float32)


_TABLE = _make_table()


def _bce_body(yh_hbm, y_hbm, tab_hbm, out_hbm,
              tab_v, yh0, y0, yh1, y1, out_v,
              sem_t, sem_a, sem_b, sem_o):
    c = lax.axis_index("c")
    s = lax.axis_index("s")
    wid = c * NS + s
    col0 = wid * COLS_W

    pltpu.make_async_copy(tab_hbm, tab_v, sem_t).start()

    bufs = ((yh0, y0), (yh1, y1))

    def start(ci, slot):
        c0 = col0 + ci * CHUNK
        yv, tv = bufs[slot]
        pltpu.make_async_copy(yh_hbm.at[:, pl.ds(c0, CHUNK)], yv, sem_a).start()
        pltpu.make_async_copy(y_hbm.at[:, pl.ds(c0, CHUNK)], tv, sem_b).start()

    def wait(ci, slot):
        c0 = col0 + ci * CHUNK
        yv, tv = bufs[slot]
        pltpu.make_async_copy(yh_hbm.at[:, pl.ds(c0, CHUNK)], yv, sem_a).wait()
        pltpu.make_async_copy(y_hbm.at[:, pl.ds(c0, CHUNK)], tv, sem_b).wait()

    start(0, 0)
    pltpu.make_async_copy(tab_hbm, tab_v, sem_t).wait()

    one = jnp.float32(1.0)

    def elem(p, t, acc, cnt):
        valid = t > -0.5
        x = jnp.where(t > 0.5, p, one - p)
        x = jnp.where(valid, x, one)
        idx = (lax.bitcast_convert_type(x, jnp.int32) >> 15) - _BIAS
        acc = acc - plsc.load_gather(tab_v, [idx])
        cnt = cnt + jnp.where(valid, one, 0.0)
        return acc, cnt

    start(1, 1)

    def half(ci, slot, carry):
        acc, cnt = carry
        wait(ci, slot)
        yv, tv = bufs[slot]

        def rows(r, carry):
            acc, cnt = carry
            for v in range(VPR):
                p = yv[r, pl.ds(v * L, L)]
                t = tv[r, pl.ds(v * L, L)]
                acc, cnt = elem(p, t, acc, cnt)
            return acc, cnt

        carry = lax.fori_loop(0, F, rows, (acc, cnt))

        @pl.when(ci + 2 < NCHUNK)
        def _():
            start(ci + 2, slot)

        return carry

    def chunk_pair(i, carry):
        ci = i * 2
        carry = half(ci, 0, carry)
        return half(ci + 1, 1, carry)

    acc, cnt = lax.fori_loop(
        0, NCHUNK // 2, chunk_pair,
        (jnp.zeros((L,), jnp.float32), jnp.zeros((L,), jnp.float32)))

    out_v[pl.ds(0, L)] = acc
    out_v[pl.ds(L, L)] = cnt
    pltpu.make_async_copy(out_v, out_hbm.at[pl.ds(wid * 2 * L, 2 * L)],
                          sem_o).start()
    pltpu.make_async_copy(out_v, out_hbm.at[pl.ds(wid * 2 * L, 2 * L)],
                          sem_o).wait()


_bce_call = pl.kernel(
    _bce_body,
    mesh=plsc.VectorSubcoreMesh(core_axis_name="c", subcore_axis_name="s", num_cores=NC),
    compiler_params=pltpu.CompilerParams(
        needs_layout_passes=False, skip_device_barrier=True),
    out_type=jax.ShapeDtypeStruct((NW * 2 * L,), jnp.float32),
    scratch_types=[
        pltpu.VMEM((_TAB_PAD,), jnp.float32),
        pltpu.VMEM((F, CHUNK), jnp.float32),
        pltpu.VMEM((F, CHUNK), jnp.float32),
        pltpu.VMEM((F, CHUNK), jnp.float32),
        pltpu.VMEM((F, CHUNK), jnp.float32),
        pltpu.VMEM((2 * L,), jnp.float32),
        pltpu.SemaphoreType.DMA,
        pltpu.SemaphoreType.DMA,
        pltpu.SemaphoreType.DMA,
        pltpu.SemaphoreType.DMA,
    ],
)


def _tc_body(yh_ref, y_ref, sum_ref, cnt_ref):
    p = yh_ref[...]
    t = y_ref[...]
    valid = t > -0.5
    x = jnp.where(t > 0.5, p, 1.0 - p)
    x = jnp.where(valid, x, 1.0)
    lnx = jnp.maximum(jnp.log(x), -100.0)

    @pl.when(pl.program_id(0) == 0)
    def _():
        sum_ref[...] = jnp.zeros((1, 1, 128), jnp.float32)
        cnt_ref[...] = jnp.zeros((1, 1, 128), jnp.float32)

    sum_ref[...] += jnp.full((1, 1, 128), -jnp.sum(lnx), jnp.float32)
    cnt_ref[...] += jnp.full((1, 1, 128), jnp.sum(jnp.where(valid, 1.0, 0.0)),
                             jnp.float32)


_tc_call = pl.pallas_call(
    _tc_body,
    grid=(TC_G,),
    in_specs=[
        pl.BlockSpec((F, TC_BLK), lambda i: (0, SC_COLS // TC_BLK + i)),
        pl.BlockSpec((F, TC_BLK), lambda i: (0, SC_COLS // TC_BLK + i)),
    ],
    out_specs=[
        pl.BlockSpec((1, 1, 128), lambda i: (0, 0, 0)),
        pl.BlockSpec((1, 1, 128), lambda i: (0, 0, 0)),
    ],
    out_shape=[
        jax.ShapeDtypeStruct((1, 1, 128), jnp.float32),
        jax.ShapeDtypeStruct((1, 1, 128), jnp.float32),
    ],
)


def kernel(y_hat, y):
    yt, tt = y_hat.T, y.T
    parts = _bce_call(yt, tt, jnp.asarray(_TABLE)).reshape(NW, 2, L)
    tc_sum, tc_cnt = _tc_call(yt, tt)
    total = jnp.sum(parts[:, 0]) + tc_sum[0, 0, 0]
    n = jnp.sum(parts[:, 1]) + tc_cnt[0, 0, 0]
    return total / n


# speedup vs baseline: 2.0915x; 2.0050x over previous
import numpy as np

import jax
import jax.numpy as jnp
from jax import lax
from jax.experimental import pallas as pl
from jax.experimental.pallas import tpu as pltpu
from jax.experimental.pallas import tpu_sc as plsc

B, F = 16384, 100
NC, NS, L = 2, 16, 16
NW = NC * NS
SC_COLS = 0
COLS_W = SC_COLS // NW
CHUNK = 128
NCHUNK = COLS_W // CHUNK
VPR = CHUNK // L
TC_BLK = 1024
TC_G = (B - SC_COLS) // TC_BLK

_BIAS = 113 << 8
_NTAB = 3585
_TAB_PAD = 3648


def _make_table() -> np.ndarray:
    i = np.arange(_NTAB - 1, dtype=np.int64)
    lo = (i + _BIAS) << 15
    hi = lo + (1 << 15)
    xlo = lo.astype(np.uint32).view(np.float32).astype(np.float64)
    xhi = hi.astype(np.uint32).view(np.float32).astype(np.float64)
    f = lambda x: x * np.log(x) - x
    tab = (f(xhi) - f(xlo)) / (xhi - xlo)
    tab = np.append(tab, 0.0)
    return np.pad(tab, (0, _TAB_PAD - _NTAB)).astype(np.float32)


_TABLE = _make_table()


def _bce_body(yh_hbm, y_hbm, tab_hbm, out_hbm,
              tab_v, yh0, y0, yh1, y1, out_v,
              sem_t, sem_a, sem_b, sem_o):
    c = lax.axis_index("c")
    s = lax.axis_index("s")
    wid = c * NS + s
    col0 = wid * COLS_W

    pltpu.make_async_copy(tab_hbm, tab_v, sem_t).start()

    bufs = ((yh0, y0), (yh1, y1))

    def start(ci, slot):
        c0 = col0 + ci * CHUNK
        yv, tv = bufs[slot]
        pltpu.make_async_copy(yh_hbm.at[:, pl.ds(c0, CHUNK)], yv, sem_a).start()
        pltpu.make_async_copy(y_hbm.at[:, pl.ds(c0, CHUNK)], tv, sem_b).start()

    def wait(ci, slot):
        c0 = col0 + ci * CHUNK
        yv, tv = bufs[slot]
        pltpu.make_async_copy(yh_hbm.at[:, pl.ds(c0, CHUNK)], yv, sem_a).wait()
        pltpu.make_async_copy(y_hbm.at[:, pl.ds(c0, CHUNK)], tv, sem_b).wait()

    start(0, 0)
    pltpu.make_async_copy(tab_hbm, tab_v, sem_t).wait()

    one = jnp.float32(1.0)

    def elem(p, t, acc, cnt):
        valid = t > -0.5
        x = jnp.where(t > 0.5, p, one - p)
        x = jnp.where(valid, x, one)
        idx = (lax.bitcast_convert_type(x, jnp.int32) >> 15) - _BIAS
        acc = acc - plsc.load_gather(tab_v, [idx])
        cnt = cnt + jnp.where(valid, one, 0.0)
        return acc, cnt

    start(1, 1)

    def half(ci, slot, carry):
        acc, cnt = carry
        wait(ci, slot)
        yv, tv = bufs[slot]

        def rows(r, carry):
            acc, cnt = carry
            for v in range(VPR):
                p = yv[r, pl.ds(v * L, L)]
                t = tv[r, pl.ds(v * L, L)]
                acc, cnt = elem(p, t, acc, cnt)
            return acc, cnt

        carry = lax.fori_loop(0, F, rows, (acc, cnt))

        @pl.when(ci + 2 < NCHUNK)
        def _():
            start(ci + 2, slot)

        return carry

    def chunk_pair(i, carry):
        ci = i * 2
        carry = half(ci, 0, carry)
        return half(ci + 1, 1, carry)

    acc, cnt = lax.fori_loop(
        0, NCHUNK // 2, chunk_pair,
        (jnp.zeros((L,), jnp.float32), jnp.zeros((L,), jnp.float32)))

    out_v[pl.ds(0, L)] = acc
    out_v[pl.ds(L, L)] = cnt
    pltpu.make_async_copy(out_v, out_hbm.at[pl.ds(wid * 2 * L, 2 * L)],
                          sem_o).start()
    pltpu.make_async_copy(out_v, out_hbm.at[pl.ds(wid * 2 * L, 2 * L)],
                          sem_o).wait()


_bce_call = pl.kernel(
    _bce_body,
    mesh=plsc.VectorSubcoreMesh(core_axis_name="c", subcore_axis_name="s", num_cores=NC),
    compiler_params=pltpu.CompilerParams(
        needs_layout_passes=False, skip_device_barrier=True),
    out_type=jax.ShapeDtypeStruct((NW * 2 * L,), jnp.float32),
    scratch_types=[
        pltpu.VMEM((_TAB_PAD,), jnp.float32),
        pltpu.VMEM((F, CHUNK), jnp.float32),
        pltpu.VMEM((F, CHUNK), jnp.float32),
        pltpu.VMEM((F, CHUNK), jnp.float32),
        pltpu.VMEM((F, CHUNK), jnp.float32),
        pltpu.VMEM((2 * L,), jnp.float32),
        pltpu.SemaphoreType.DMA,
        pltpu.SemaphoreType.DMA,
        pltpu.SemaphoreType.DMA,
        pltpu.SemaphoreType.DMA,
    ],
)


def _tc_body(yh_ref, y_ref, sum_ref, cnt_ref):
    p = yh_ref[...]
    t = y_ref[...]
    valid = t > -0.5
    x = jnp.where(t > 0.5, p, 1.0 - p)
    x = jnp.where(valid, x, 1.0)
    lnx = jnp.maximum(jnp.log(x), -100.0)

    @pl.when(pl.program_id(0) == 0)
    def _():
        sum_ref[...] = jnp.zeros((1, 1, 128), jnp.float32)
        cnt_ref[...] = jnp.zeros((1, 1, 128), jnp.float32)

    sum_ref[...] += jnp.full((1, 1, 128), -jnp.sum(lnx), jnp.float32)
    cnt_ref[...] += jnp.full((1, 1, 128), jnp.sum(jnp.where(valid, 1.0, 0.0)),
                             jnp.float32)


_tc_call = pl.pallas_call(
    _tc_body,
    grid=(TC_G,),
    in_specs=[
        pl.BlockSpec((F, TC_BLK), lambda i: (0, SC_COLS // TC_BLK + i)),
        pl.BlockSpec((F, TC_BLK), lambda i: (0, SC_COLS // TC_BLK + i)),
    ],
    out_specs=[
        pl.BlockSpec((1, 1, 128), lambda i: (0, 0, 0)),
        pl.BlockSpec((1, 1, 128), lambda i: (0, 0, 0)),
    ],
    out_shape=[
        jax.ShapeDtypeStruct((1, 1, 128), jnp.float32),
        jax.ShapeDtypeStruct((1, 1, 128), jnp.float32),
    ],
)


def kernel(y_hat, y):
    yt, tt = y_hat.T, y.T
    tc_sum, tc_cnt = _tc_call(yt, tt)
    return tc_sum[0, 0, 0] / tc_cnt[0, 0, 0]
